# baseline (speedup 1.0000x reference)
import jax
import jax.numpy as jnp
from jax import lax
from jax.experimental import pallas as pl
from jax.experimental.pallas import tpu as pltpu
from jax.experimental.pallas import tpu_sc as plsc

N_VOCAB = 100000
N_EMBD = 64
N_TOKEN = 200
BATCH = 4096

NC = 2
NS = 16
NW = NC * NS
BBLK = BATCH // NW
LANES = 16
VPR = N_EMBD // LANES
NBUF = 4


def _emb_kernel(table_hbm, idx_hbm, pos_hbm, out_hbm,
                idx_v, pos_v, bufs, buf2s, gsems, wsems):
    wid = lax.axis_index("s") * NC + lax.axis_index("c")
    col0 = wid * BBLK

    pltpu.sync_copy(idx_hbm.at[:, pl.ds(col0, BBLK)], idx_v)
    pltpu.sync_copy(pos_hbm, pos_v)

    def gather(t, b):
        return pltpu.make_async_copy(
            table_hbm.at[idx_v.at[t]], bufs[b], gsems[b])

    def write(t, b):
        return pltpu.make_async_copy(
            buf2s[b], out_hbm.at[t, :, pl.ds(col0, BBLK)], wsems[b])

    for b in range(NBUF):
        gather(b, b).start()

    rows = [c * LANES + lax.iota(jnp.int32, LANES) for c in range(VPR)]

    def outer(k, carry):
        i = k * NBUF
        for b in range(NBUF):
            t = i + b
            gather(t, b).wait()

            @pl.when(t >= NBUF)
            def _():
                write(t - NBUF, b).wait()

            pv = [pos_v[t, pl.ds(c * LANES, LANES)] for c in range(VPR)]

            def tr_body(bb, c2):
                cols = jnp.full((LANES,), bb, jnp.int32)
                for c in range(VPR):
                    v = bufs[b][bb, pl.ds(c * LANES, LANES)] + pv[c]
                    plsc.store_scatter(buf2s[b], [rows[c], cols], v)
                return c2
            lax.fori_loop(0, BBLK, tr_body, 0, unroll=2)

            write(t, b).start()

            @pl.when(t + NBUF < N_TOKEN)
            def _():
                gather(t + NBUF, b).start()
        return carry

    lax.fori_loop(0, N_TOKEN // NBUF, outer, 0)

    for b in range(NBUF):
        write(N_TOKEN - NBUF + b, b).wait()


def _emb(table, idx_t, pos):
    mesh = plsc.VectorSubcoreMesh(core_axis_name="c", subcore_axis_name="s")
    f = pl.kernel(
        _emb_kernel,
        out_type=jax.ShapeDtypeStruct((N_TOKEN, N_EMBD, BATCH), jnp.float32),
        mesh=mesh,
        scratch_types=[
            pltpu.VMEM((N_TOKEN, BBLK), jnp.int32),
            pltpu.VMEM((N_TOKEN, N_EMBD), jnp.float32),
            [pltpu.VMEM((BBLK, N_EMBD), jnp.float32) for _ in range(NBUF)],
            [pltpu.VMEM((N_EMBD, BBLK), jnp.float32) for _ in range(NBUF)],
            [pltpu.SemaphoreType.DMA for _ in range(NBUF)],
            [pltpu.SemaphoreType.DMA for _ in range(NBUF)],
        ],
        compiler_params=pltpu.CompilerParams(
            use_tc_tiling_on_sc=False, needs_layout_passes=False),
    )
    return f(table, idx_t, pos)


def kernel(tokens, token_embedding, position_embedding):
    idx_t = tokens.T
    out_t = _emb(token_embedding, idx_t, position_embedding)
    return out_t.transpose(2, 0, 1)


# speedup vs baseline: 1.2897x; 1.2897x over previous
import jax
import jax.numpy as jnp
from jax import lax
from jax.experimental import pallas as pl
from jax.experimental.pallas import tpu as pltpu
from jax.experimental.pallas import tpu_sc as plsc

N_VOCAB = 100000
N_EMBD = 64
N_TOKEN = 200
BATCH = 4096

NC = 2
NS = 16
NW = NC * NS
B_FLAT = BATCH * N_TOKEN
B_PER_W = B_FLAT // NW
SEQ_PER_W = B_PER_W // N_TOKEN
LANES = 16
VPR = N_EMBD // LANES
HROW = N_TOKEN // 2
NBUF = 2


def _emb_kernel(table_hbm, idx_hbm, pos_hbm, out_hbm,
                idx_v, pos_v, bufs, buf2s, gsems, wsems):
    wid = lax.axis_index("s") * NC + lax.axis_index("c")
    ibase = wid * B_PER_W
    obase = wid * SEQ_PER_W * HROW

    pltpu.sync_copy(idx_hbm.at[pl.ds(ibase, B_PER_W)], idx_v)
    pltpu.sync_copy(pos_hbm, pos_v)

    def gather_a(s, b):
        return pltpu.make_async_copy(
            table_hbm.at[idx_v.at[pl.ds(s * N_TOKEN, 128)]],
            bufs[b].at[pl.ds(0, 128)], gsems[b])

    def gather_b(s, b):
        return pltpu.make_async_copy(
            table_hbm.at[idx_v.at[pl.ds(s * N_TOKEN + 128, 72)]],
            bufs[b].at[pl.ds(128, 72)], gsems[b])

    def write(s, b):
        return pltpu.make_async_copy(
            buf2s[b], out_hbm.at[pl.ds(obase + s * HROW, HROW)], wsems[b])

    for b in range(NBUF):
        gather_a(b, b).start()
        gather_b(b, b).start()

    def outer(k, carry):
        i = k * NBUF
        for b in range(NBUF):
            s = i + b
            gather_a(s, b).wait()
            gather_b(s, b).wait()

            @pl.when(s >= NBUF)
            def _():
                write(s - NBUF, b).wait()

            def add_body(p, c2):
                for sub in (0, 1):
                    for c in range(VPR):
                        sl = pl.ds(c * LANES, LANES)
                        dsl = pl.ds(sub * N_EMBD + c * LANES, LANES)
                        buf2s[b][p, dsl] = (bufs[b][2 * p + sub, sl]
                                            + pos_v[2 * p + sub, sl])
                return c2
            lax.fori_loop(0, HROW, add_body, 0, unroll=2)

            write(s, b).start()

            @pl.when(s + NBUF < SEQ_PER_W)
            def _():
                gather_a(s + NBUF, b).start()
                gather_b(s + NBUF, b).start()
        return carry

    lax.fori_loop(0, SEQ_PER_W // NBUF, outer, 0)

    for b in range(NBUF):
        write(SEQ_PER_W - NBUF + b, b).wait()


def _emb(table, idx_flat, pos):
    mesh = plsc.VectorSubcoreMesh(core_axis_name="c", subcore_axis_name="s")
    f = pl.kernel(
        _emb_kernel,
        out_type=jax.ShapeDtypeStruct((B_FLAT // 2, 2 * N_EMBD), jnp.float32),
        mesh=mesh,
        scratch_types=[
            pltpu.VMEM((B_PER_W,), jnp.int32),
            pltpu.VMEM((N_TOKEN, N_EMBD), jnp.float32),
            [pltpu.VMEM((N_TOKEN, N_EMBD), jnp.float32) for _ in range(NBUF)],
            [pltpu.VMEM((HROW, 2 * N_EMBD), jnp.float32) for _ in range(NBUF)],
            [pltpu.SemaphoreType.DMA for _ in range(NBUF)],
            [pltpu.SemaphoreType.DMA for _ in range(NBUF)],
        ],
        compiler_params=pltpu.CompilerParams(
            use_tc_tiling_on_sc=False, needs_layout_passes=False),
    )
    return f(table, idx_flat, pos)


def kernel(tokens, token_embedding, position_embedding):
    idx_flat = tokens.reshape(B_FLAT)
    out = _emb(token_embedding, idx_flat, position_embedding)
    return out.reshape(BATCH, N_TOKEN, N_EMBD)
